# baseline (speedup 1.0000x reference)
import functools

import jax
import jax.numpy as jnp
from jax import lax
from jax.experimental import pallas as pl
from jax.experimental.pallas import tpu as pltpu
from jax.experimental.pallas import tpu_sc as plsc

_NC = 2
_NS = 16
_NW = _NC * _NS
_LANES = 16


@functools.cache
def _build(bz, nz, vocab, dim):
  n = bz * nz
  rw = n // _NW
  assert n % _NW == 0 and rw % nz == 0 and nz % 2 == 0
  reps = 2
  c = reps * nz
  nchunk = rw // c
  assert rw % c == 0 and c % 8 == 0 and nchunk % 2 == 0
  nreg = dim // _LANES

  mesh = plsc.VectorSubcoreMesh(core_axis_name="c", subcore_axis_name="s")

  @functools.partial(
      pl.kernel,
      out_type=jax.ShapeDtypeStruct((n // 2, 2 * dim), jnp.float32),
      mesh=mesh,
      compiler_params=pltpu.CompilerParams(use_tc_tiling_on_sc=False),
      scratch_types=[
          pltpu.VMEM((nz * dim,), jnp.float32),
          pltpu.VMEM((2, c), jnp.int32),
          pltpu.VMEM((2, c, dim), jnp.float32),
          pltpu.VMEM((c // 2, 2 * dim), jnp.float32),
          pltpu.SemaphoreType.DMA,
          pltpu.SemaphoreType.DMA,
      ],
  )
  def k(seq_hbm, tok_hbm, pos_hbm, out_hbm, pos_v, idx_v, rows_v, stage_v,
        sem0, sem1):
    sems = (sem0, sem1)
    wid = lax.axis_index("s") * _NC + lax.axis_index("c")
    base_w = wid * rw
    pltpu.sync_copy(pos_hbm, pos_v)

    for b in range(2):
      pltpu.sync_copy(seq_hbm.at[pl.ds(base_w + b * c, c)], idx_v.at[b])
      pltpu.async_copy(tok_hbm.at[idx_v.at[b]], rows_v.at[b], sems[b])

    def group(gg, carry):
      for b in range(2):
        g = gg * 2 + b
        pltpu.make_async_copy(
            tok_hbm.at[idx_v.at[b]], rows_v.at[b], sems[b]).wait()

        def kp_body(kp, c2):
          p0 = 2 * kp
          pv0 = [pos_v[pl.ds(p0 * dim + j * _LANES, _LANES)]
                 for j in range(nreg)]
          pv1 = [pos_v[pl.ds(p0 * dim + dim + j * _LANES, _LANES)]
                 for j in range(nreg)]
          for r in range(reps):
            i0 = r * nz + p0
            kl = r * (nz // 2) + kp
            for j in range(nreg):
              stage_v[kl, pl.ds(j * _LANES, _LANES)] = (
                  rows_v[b, i0, pl.ds(j * _LANES, _LANES)] + pv0[j])
            for j in range(nreg):
              stage_v[kl, pl.ds(dim + j * _LANES, _LANES)] = (
                  rows_v[b, i0 + 1, pl.ds(j * _LANES, _LANES)] + pv1[j])
          return c2

        lax.fori_loop(0, nz // 2, kp_body, 0)
        pltpu.sync_copy(
            stage_v, out_hbm.at[pl.ds((base_w + g * c) // 2, c // 2)])

        @pl.when(g + 2 < nchunk)
        def _():
          pltpu.sync_copy(
              seq_hbm.at[pl.ds(base_w + (g + 2) * c, c)], idx_v.at[b])
          pltpu.async_copy(tok_hbm.at[idx_v.at[b]], rows_v.at[b], sems[b])

      return carry

    lax.fori_loop(0, nchunk // 2, group, 0)

  return k


def kernel(sequence, tok_embeds, pos_embeds):
  bz, nz = sequence.shape
  vocab, dim = tok_embeds.shape
  seq_flat = sequence.reshape(-1).astype(jnp.int32)
  pos_flat = pos_embeds[:nz].reshape(-1)
  out = _build(bz, nz, vocab, dim)(seq_flat, tok_embeds, pos_flat)
  return out.reshape(bz, nz, dim)


# speedup vs baseline: 1.0264x; 1.0264x over previous
import functools

import jax
import jax.numpy as jnp
from jax import lax
from jax.experimental import pallas as pl
from jax.experimental.pallas import tpu as pltpu
from jax.experimental.pallas import tpu_sc as plsc

_NC = 2
_NS = 16
_NW = _NC * _NS
_LANES = 16


@functools.cache
def _build(bz, nz, vocab, dim):
  n = bz * nz
  rw = n // _NW
  assert n % _NW == 0 and rw % nz == 0
  reps = 4
  c = reps * nz
  nchunk = rw // c
  assert rw % c == 0 and c % 8 == 0 and nchunk % 2 == 0
  nreg = dim // _LANES

  mesh = plsc.VectorSubcoreMesh(core_axis_name="c", subcore_axis_name="s")

  @functools.partial(
      pl.kernel,
      out_type=jax.ShapeDtypeStruct((n, dim), jnp.float32),
      mesh=mesh,
      compiler_params=pltpu.CompilerParams(use_tc_tiling_on_sc=False),
      scratch_types=[
          pltpu.VMEM((nz, dim), jnp.float32),
          pltpu.VMEM((2, c), jnp.int32),
          pltpu.VMEM((2, c, dim), jnp.float32),
          pltpu.SemaphoreType.DMA,
          pltpu.SemaphoreType.DMA,
      ],
  )
  def k(seq_hbm, tok_hbm, pos_hbm, out_hbm, pos_v, idx_v, rows_v, sem0, sem1):
    sems = (sem0, sem1)
    wid = lax.axis_index("s") * _NC + lax.axis_index("c")
    base_w = wid * rw
    pltpu.sync_copy(pos_hbm.at[pl.ds(0, nz)], pos_v)

    for b in range(2):
      pltpu.sync_copy(seq_hbm.at[pl.ds(base_w + b * c, c)], idx_v.at[b])
      pltpu.async_copy(tok_hbm.at[idx_v.at[b]], rows_v.at[b], sems[b])

    def group(gg, carry):
      for b in range(2):
        g = gg * 2 + b
        pltpu.make_async_copy(
            tok_hbm.at[idx_v.at[b]], rows_v.at[b], sems[b]).wait()

        def p_body(p, c2):
          for j in range(nreg):
            pv = pos_v[p, pl.ds(j * _LANES, _LANES)]
            for r in range(reps):
              plsc.addupdate(
                  rows_v.at[b, r * nz + p, pl.ds(j * _LANES, _LANES)], pv)
          return c2

        lax.fori_loop(0, nz, p_body, 0)
        pltpu.sync_copy(rows_v.at[b], out_hbm.at[pl.ds(base_w + g * c, c)])

        @pl.when(g + 2 < nchunk)
        def _():
          pltpu.sync_copy(
              seq_hbm.at[pl.ds(base_w + (g + 2) * c, c)], idx_v.at[b])
          pltpu.async_copy(tok_hbm.at[idx_v.at[b]], rows_v.at[b], sems[b])

      return carry

    lax.fori_loop(0, nchunk // 2, group, 0)

  return k


def kernel(sequence, tok_embeds, pos_embeds):
  bz, nz = sequence.shape
  vocab, dim = tok_embeds.shape
  seq_flat = sequence.reshape(-1).astype(jnp.int32)
  out = _build(bz, nz, vocab, dim)(seq_flat, tok_embeds, pos_embeds)
  return out.reshape(bz, nz, dim)


# speedup vs baseline: 1.0368x; 1.0102x over previous
import functools

import jax
import jax.numpy as jnp
from jax import lax
from jax.experimental import pallas as pl
from jax.experimental.pallas import tpu as pltpu
from jax.experimental.pallas import tpu_sc as plsc

_NC = 2
_NS = 16
_NW = _NC * _NS
_LANES = 16
_NBUF = 4


@functools.cache
def _build(bz, nz, vocab, dim):
  n = bz * nz
  rw = n // _NW
  assert n % _NW == 0 and rw % nz == 0
  reps = 2
  c = reps * nz
  nchunk = rw // c
  assert rw % c == 0 and c % 8 == 0 and nchunk % _NBUF == 0
  nreg = dim // _LANES

  mesh = plsc.VectorSubcoreMesh(core_axis_name="c", subcore_axis_name="s")

  @functools.partial(
      pl.kernel,
      out_type=jax.ShapeDtypeStruct((n, dim), jnp.float32),
      mesh=mesh,
      compiler_params=pltpu.CompilerParams(use_tc_tiling_on_sc=False),
      scratch_types=[
          pltpu.VMEM((nz, dim), jnp.float32),
          pltpu.VMEM((_NBUF, c), jnp.int32),
          pltpu.VMEM((_NBUF, c, dim), jnp.float32),
      ] + [pltpu.SemaphoreType.DMA] * (2 * _NBUF),
  )
  def k(seq_hbm, tok_hbm, pos_hbm, out_hbm, pos_v, idx_v, rows_v, *sems):
    gsem = sems[:_NBUF]
    ssem = sems[_NBUF:]
    wid = lax.axis_index("s") * _NC + lax.axis_index("c")
    base_w = wid * rw
    pltpu.sync_copy(pos_hbm.at[pl.ds(0, nz)], pos_v)

    for b in range(2):
      pltpu.sync_copy(seq_hbm.at[pl.ds(base_w + b * c, c)], idx_v.at[b])
      pltpu.async_copy(tok_hbm.at[idx_v.at[b]], rows_v.at[b], gsem[b])

    def group(gg, carry):
      for b4 in range(_NBUF):
        g = gg * _NBUF + b4
        pltpu.make_async_copy(
            tok_hbm.at[idx_v.at[b4]], rows_v.at[b4], gsem[b4]).wait()

        def p_body(p, c2):
          for j in range(nreg):
            pv = pos_v[p, pl.ds(j * _LANES, _LANES)]
            for r in range(reps):
              plsc.addupdate(
                  rows_v.at[b4, r * nz + p, pl.ds(j * _LANES, _LANES)], pv)
          return c2

        lax.fori_loop(0, nz, p_body, 0)
        pltpu.async_copy(
            rows_v.at[b4], out_hbm.at[pl.ds(base_w + g * c, c)], ssem[b4])

        bp = (b4 + 2) % _NBUF

        @pl.when(g + 2 < nchunk)
        def _():
          @pl.when(g >= 2)
          def _():
            pltpu.make_async_copy(
                rows_v.at[bp], out_hbm.at[pl.ds(base_w, c)], ssem[bp]).wait()

          pltpu.sync_copy(
              seq_hbm.at[pl.ds(base_w + (g + 2) * c, c)], idx_v.at[bp])
          pltpu.async_copy(tok_hbm.at[idx_v.at[bp]], rows_v.at[bp], gsem[bp])

      return carry

    lax.fori_loop(0, nchunk // _NBUF, group, 0)

    for b in range(_NBUF):
      pltpu.make_async_copy(
          rows_v.at[b], out_hbm.at[pl.ds(base_w, c)], ssem[b]).wait()

  return k


def kernel(sequence, tok_embeds, pos_embeds):
  bz, nz = sequence.shape
  vocab, dim = tok_embeds.shape
  seq_flat = sequence.reshape(-1).astype(jnp.int32)
  out = _build(bz, nz, vocab, dim)(seq_flat, tok_embeds, pos_embeds)
  return out.reshape(bz, nz, dim)
